# Initial kernel scaffold; baseline (speedup 1.0000x reference)
import jax
import jax.numpy as jnp
from jax.experimental import pallas as pl


def kernel(hidden_states, gate_w, w1, w3, w2):
    raise NotImplementedError("write your pallas kernel here")


# speedup vs baseline: 1.4098x; 1.4098x over previous
import functools

import jax
import jax.numpy as jnp
from jax import lax
from jax.experimental import pallas as pl
from jax.experimental.pallas import tpu as pltpu
from jax.experimental.pallas import tpu_sc as plsc

T = 2048
D = 2048
E = 8
F = 4096
K = 2
A = T * K

BM = 256
NB = 24
PADDED = NB * BM
BF = 512
NF = F // BF


def _router_body(x_ref, gw_ref, slot_ref, wa_ref, ends_ref):
    x = x_ref[...]
    gw = gw_ref[...]
    logits = lax.dot_general(x, gw, (((1,), (1,)), ((), ())),
                             preferred_element_type=jnp.float32)

    iota_e = lax.broadcasted_iota(jnp.int32, (T, E), 1)
    m1 = jnp.max(logits, axis=1, keepdims=True)
    i1 = jnp.min(jnp.where(logits == m1, iota_e, E), axis=1, keepdims=True)
    masked = jnp.where(iota_e == i1, -jnp.inf, logits)
    m2 = jnp.max(masked, axis=1, keepdims=True)
    i2 = jnp.min(jnp.where(masked == m2, iota_e, E), axis=1, keepdims=True)

    ed = jnp.exp(m2 - m1)
    w1v = 1.0 / (1.0 + ed)
    w2v = 1.0 - w1v

    oh1 = (iota_e == i1).astype(jnp.float32)
    oh2 = (iota_e == i2).astype(jnp.float32)
    oh = jnp.concatenate([oh1, oh2], axis=0)

    CH = 1024
    tri = (lax.broadcasted_iota(jnp.int32, (CH, CH), 0)
           >= lax.broadcasted_iota(jnp.int32, (CH, CH), 1)).astype(jnp.float32)
    carry = jnp.zeros((1, E), jnp.float32)
    cums = []
    for i in range(A // CH):
        blk = oh[i * CH:(i + 1) * CH, :]
        c = lax.dot_general(tri, blk, (((1,), (0,)), ((), ())),
                            precision=lax.Precision.HIGHEST,
                            preferred_element_type=jnp.float32) + carry
        c = jnp.round(c)
        carry = c[CH - 1:CH, :]
        cums.append(c)
    cum = jnp.concatenate(cums, axis=0)
    counts = cum[A - 1:A, :]

    nb = jnp.floor((counts + (BM - 1)) / BM)
    mlt = (lax.broadcasted_iota(jnp.int32, (E, E), 0)
           < lax.broadcasted_iota(jnp.int32, (E, E), 1)).astype(jnp.float32)
    s = jnp.round(lax.dot_general(nb, mlt, (((1,), (0,)), ((), ())),
                                  precision=lax.Precision.HIGHEST,
                                  preferred_element_type=jnp.float32))
    ends = s + nb
    off_rows = BM * s

    rank = jnp.sum(oh * cum, axis=1, keepdims=True) - 1.0
    base = jnp.sum(oh * off_rows, axis=1, keepdims=True)
    slot = (base + rank).astype(jnp.int32)

    wa = jnp.concatenate([w1v, w2v], axis=0)

    slot_ref[...] = slot
    wa_ref[...] = wa
    ends_ref[...] = ends.astype(jnp.int32)


def _router(x, gate_w):
    return pl.pallas_call(
        _router_body,
        out_shape=(
            jax.ShapeDtypeStruct((A, 1), jnp.int32),
            jax.ShapeDtypeStruct((A, 1), jnp.float32),
            jax.ShapeDtypeStruct((1, E), jnp.int32),
        ),
    )(x, gate_w)


_NC = 2
_NS = 16
_NW = _NC * _NS

_S2_PER_W = A // _NW
_S2_CHUNK = 32
_S2_NCH = _S2_PER_W // _S2_CHUNK


def _scatter_body(x_hbm, slot_hbm, wa_hbm, xs_hbm, ws_hbm,
                  xbuf, idxbuf, wsv, svbuf, wvbuf, sem):
    wid = lax.axis_index("s") * _NC + lax.axis_index("c")
    a0 = wid * _S2_PER_W
    t0 = a0 % T
    for c in range(_S2_NCH):
        pltpu.sync_copy(x_hbm.at[pl.ds(t0 + c * _S2_CHUNK, _S2_CHUNK)], xbuf)
        pltpu.sync_copy(slot_hbm.at[pl.ds(a0 + c * _S2_CHUNK, _S2_CHUNK)],
                        idxbuf)
        pltpu.async_copy(xbuf, xs_hbm.at[idxbuf], sem).wait()

    @pl.when(wid == 0)
    def _():
        def zero_body(i, _):
            wsv[pl.ds(i * 16, 16)] = jnp.zeros((16,), jnp.float32)
            return 0
        lax.fori_loop(0, PADDED // 16, zero_body, 0)
        pltpu.sync_copy(slot_hbm, svbuf)
        pltpu.sync_copy(wa_hbm, wvbuf)

        def scat_body(i, _):
            sl = svbuf[pl.ds(i * 16, 16)]
            wv = wvbuf[pl.ds(i * 16, 16)]
            plsc.store_scatter(wsv, [sl], wv)
            return 0
        lax.fori_loop(0, A // 16, scat_body, 0)
        pltpu.sync_copy(wsv, ws_hbm)


def _scatter(x, slot, wa):
    kfn = functools.partial(
        pl.kernel,
        mesh=plsc.VectorSubcoreMesh(core_axis_name="c", subcore_axis_name="s"),
        out_type=(
            jax.ShapeDtypeStruct((PADDED, D), jnp.float32),
            jax.ShapeDtypeStruct((PADDED,), jnp.float32),
        ),
        scratch_types=[
            pltpu.VMEM((_S2_CHUNK, D), jnp.float32),
            pltpu.VMEM((_S2_CHUNK,), jnp.int32),
            pltpu.VMEM((PADDED,), jnp.float32),
            pltpu.VMEM((A,), jnp.int32),
            pltpu.VMEM((A,), jnp.float32),
            pltpu.SemaphoreType.DMA,
        ],
        compiler_params=pltpu.CompilerParams(needs_layout_passes=False),
    )
    return kfn(_scatter_body)(x, slot, wa)


def _block_expert(ends_ref, bc):
    e = jnp.int32(0)
    for j in range(E):
        e = e + (ends_ref[0, j] <= bc).astype(jnp.int32)
    return e


def _gmm_body(ends_ref, xs_ref, ws_ref, w1_ref, w3_ref, w2_ref, ys_ref,
              acc_ref):
    b = pl.program_id(0)
    f = pl.program_id(1)
    used = ends_ref[0, E - 1]

    @pl.when(b < used)
    def _():
        xb = xs_ref[...]
        g = lax.dot_general(xb, w1_ref[0], (((1,), (1,)), ((), ())),
                            preferred_element_type=jnp.float32)
        u = lax.dot_general(xb, w3_ref[0], (((1,), (1,)), ((), ())),
                            preferred_element_type=jnp.float32)
        h = g * lax.logistic(g) * u
        yb = lax.dot_general(h, w2_ref[0], (((1,), (1,)), ((), ())),
                             preferred_element_type=jnp.float32)

        @pl.when(f == 0)
        def _():
            acc_ref[...] = yb

        @pl.when(f != 0)
        def _():
            acc_ref[...] += yb

        @pl.when(f == NF - 1)
        def _():
            ys_ref[...] = acc_ref[...] * ws_ref[...]


def _gmm(ends, xs, ws2d, w1, w3, w2):
    def im_rows(b, f, ends_ref):
        return (jnp.minimum(b, ends_ref[0, E - 1] - 1), 0)

    def im_w13(b, f, ends_ref):
        bc = jnp.minimum(b, ends_ref[0, E - 1] - 1)
        return (_block_expert(ends_ref, bc), f, 0)

    def im_w2(b, f, ends_ref):
        bc = jnp.minimum(b, ends_ref[0, E - 1] - 1)
        return (_block_expert(ends_ref, bc), 0, f)

    grid_spec = pltpu.PrefetchScalarGridSpec(
        num_scalar_prefetch=1,
        grid=(NB, NF),
        in_specs=[
            pl.BlockSpec((BM, D), im_rows),
            pl.BlockSpec((BM, 1), im_rows),
            pl.BlockSpec((1, BF, D), im_w13),
            pl.BlockSpec((1, BF, D), im_w13),
            pl.BlockSpec((1, D, BF), im_w2),
        ],
        out_specs=pl.BlockSpec((BM, D), im_rows),
        scratch_shapes=[pltpu.VMEM((BM, D), jnp.float32)],
    )
    return pl.pallas_call(
        _gmm_body,
        grid_spec=grid_spec,
        out_shape=jax.ShapeDtypeStruct((PADDED, D), jnp.float32),
        compiler_params=pltpu.CompilerParams(
            dimension_semantics=("arbitrary", "arbitrary")),
    )(ends, xs, ws2d, w1, w3, w2)


_S4_PER_W = T // _NW
_S4_CHUNK = 16
_S4_NCH = _S4_PER_W // _S4_CHUNK


def _combine_body(ys_hbm, slot_hbm, out_hbm, idx0, idx1, r0, r1, sem0, sem1):
    wid = lax.axis_index("s") * _NC + lax.axis_index("c")
    t0 = wid * _S4_PER_W
    for c in range(_S4_NCH):
        tc0 = t0 + c * _S4_CHUNK
        pltpu.sync_copy(slot_hbm.at[pl.ds(tc0, _S4_CHUNK)], idx0)
        pltpu.sync_copy(slot_hbm.at[pl.ds(T + tc0, _S4_CHUNK)], idx1)
        cp0 = pltpu.async_copy(ys_hbm.at[idx0], r0, sem0)
        cp1 = pltpu.async_copy(ys_hbm.at[idx1], r1, sem1)
        cp0.wait()
        cp1.wait()

        def row_body(r, _):
            def col_body(j, _):
                v = r0[r, pl.ds(j * 16, 16)] + r1[r, pl.ds(j * 16, 16)]
                r0[r, pl.ds(j * 16, 16)] = v
                return 0
            lax.fori_loop(0, D // 16, col_body, 0)
            return 0
        lax.fori_loop(0, _S4_CHUNK, row_body, 0)
        pltpu.sync_copy(r0, out_hbm.at[pl.ds(tc0, _S4_CHUNK)])


def _combine(ys, slot):
    kfn = functools.partial(
        pl.kernel,
        mesh=plsc.VectorSubcoreMesh(core_axis_name="c", subcore_axis_name="s"),
        out_type=jax.ShapeDtypeStruct((T, D), jnp.float32),
        scratch_types=[
            pltpu.VMEM((_S4_CHUNK,), jnp.int32),
            pltpu.VMEM((_S4_CHUNK,), jnp.int32),
            pltpu.VMEM((_S4_CHUNK, D), jnp.float32),
            pltpu.VMEM((_S4_CHUNK, D), jnp.float32),
            pltpu.SemaphoreType.DMA,
            pltpu.SemaphoreType.DMA,
        ],
        compiler_params=pltpu.CompilerParams(needs_layout_passes=False),
    )
    return kfn(_combine_body)(ys, slot)


def kernel(hidden_states, gate_w, w1, w3, w2):
    input_shape = hidden_states.shape
    x = hidden_states.reshape(T, D)
    slot, wa, ends = _router(x, gate_w)
    slot = slot.reshape(A)
    wa = wa.reshape(A)
    xs, ws = _scatter(x, slot, wa)
    ys = _gmm(ends, xs, ws.reshape(PADDED, 1), w1, w3, w2)
    out = _combine(ys, slot)
    return out.reshape(input_shape)
